# Initial kernel scaffold; baseline (speedup 1.0000x reference)
import jax
import jax.numpy as jnp
from jax.experimental import pallas as pl


def kernel(x, src, dst, W, b):
    raise NotImplementedError("write your pallas kernel here")


# speedup vs baseline: 51.5037x; 51.5037x over previous
import functools

import jax
import jax.numpy as jnp
from jax import lax
from jax.experimental import pallas as pl
from jax.experimental.pallas import tpu as pltpu
from jax.experimental.pallas import tpu_sc as plsc

NC = 2
NS = 16
LANES = 16
CH = 80
ZR = 128


def _make_deg(E, N):
    ept = E // (NC * NS)
    n_chunks = ept // CH
    npt = N // NS
    nz = npt // ZR
    mesh = plsc.VectorSubcoreMesh(core_axis_name="c", subcore_axis_name="s",
                                  num_cores=NC, num_subcores=NS)

    @functools.partial(
        pl.kernel,
        out_type=jax.ShapeDtypeStruct((NC * N, LANES), jnp.float32),
        mesh=mesh,
        scratch_types=[
            pltpu.VMEM((CH,), jnp.int32),
            pltpu.VMEM((CH, LANES), jnp.float32),
            pltpu.VMEM((ZR, LANES), jnp.float32),
            pltpu.VMEM_SHARED((N, LANES), jnp.float32),
        ],
    )
    def deg_k(dst_hbm, out_hbm, dbuf, ones_v, zeros_v, acc_sh):
        c = lax.axis_index("c")
        s = lax.axis_index("s")
        wid = c * NS + s
        one16 = jnp.ones((LANES,), jnp.float32)
        zero16 = jnp.zeros((LANES,), jnp.float32)
        for i in range(CH):
            ones_v[i, :] = one16
        for i in range(ZR):
            zeros_v[i, :] = zero16

        rbase = s * npt

        def zbody(j, carry):
            pltpu.sync_copy(zeros_v, acc_sh.at[pl.ds(rbase + j * ZR, ZR)])
            return carry
        lax.fori_loop(0, nz, zbody, 0)
        plsc.subcore_barrier()

        ebase = wid * ept

        def ebody(g, carry):
            pltpu.sync_copy(dst_hbm.at[pl.ds(ebase + g * CH, CH)], dbuf)
            pltpu.sync_copy(ones_v, acc_sh.at[dbuf], add=True)
            return carry
        lax.fori_loop(0, n_chunks, ebody, 0)
        plsc.subcore_barrier()

        def obody(j, carry):
            r = rbase + j * ZR
            pltpu.sync_copy(acc_sh.at[pl.ds(r, ZR)],
                            out_hbm.at[pl.ds(c * N + r, ZR)])
            return carry
        lax.fori_loop(0, nz, obody, 0)

    return deg_k


def _make_agg(E, N, F):
    ept = E // NS
    n_chunks = ept // CH
    npt = N // NS
    no = npt // ZR
    mesh = plsc.VectorSubcoreMesh(core_axis_name="c", subcore_axis_name="s",
                                  num_cores=NC, num_subcores=NS)

    @functools.partial(
        pl.kernel,
        out_type=jax.ShapeDtypeStruct((NC * N, F), jnp.float32),
        mesh=mesh,
        scratch_types=[
            pltpu.VMEM((CH,), jnp.int32),
            pltpu.VMEM((CH,), jnp.int32),
            pltpu.VMEM((CH,), jnp.int32),
            pltpu.VMEM((CH, F), jnp.float32),
            pltpu.VMEM((ZR, F), jnp.float32),
            pltpu.VMEM_SHARED((N, F), jnp.float32),
            pltpu.SemaphoreType.DMA,
        ],
    )
    def agg_k(y2_hbm, src_hbm, dst_hbm, out_hbm,
              dbuf, gbuf, sbuf, rows_v, bounce_v, acc_sh, sem):
        c = lax.axis_index("c")
        s = lax.axis_index("s")
        rbase = s * npt
        off = c * N

        def ibody(j, carry):
            r = rbase + j * ZR
            pltpu.sync_copy(y2_hbm.at[pl.ds(off + r, ZR)], bounce_v)
            pltpu.sync_copy(bounce_v, acc_sh.at[pl.ds(r, ZR)])
            return carry
        lax.fori_loop(0, no, ibody, 0)
        plsc.subcore_barrier()

        ebase = s * ept

        def ebody(g, carry):
            e0 = ebase + g * CH
            pltpu.sync_copy(dst_hbm.at[pl.ds(e0, CH)], dbuf)
            pltpu.sync_copy(src_hbm.at[pl.ds(e0, CH)], sbuf)
            for i in range(CH // LANES):
                sl = pl.ds(i * LANES, LANES)
                gbuf[sl] = dbuf[sl] + off
            pltpu.async_copy(y2_hbm.at[gbuf], rows_v, sem).wait()
            pltpu.sync_copy(rows_v, acc_sh.at[sbuf], add=True)
            return carry
        lax.fori_loop(0, n_chunks, ebody, 0)
        plsc.subcore_barrier()

        def obody(j, carry):
            r = rbase + j * ZR
            pltpu.sync_copy(acc_sh.at[pl.ds(r, ZR)], bounce_v)
            pltpu.sync_copy(bounce_v, out_hbm.at[pl.ds(off + r, ZR)])
            return carry
        lax.fori_loop(0, no, obody, 0)

    return agg_k


def _prep_body(x_ref, w_ref, deg_ref, y2_ref, d_ref):
    deg = deg_ref[0] + deg_ref[1] + 1.0
    dfull = lax.rsqrt(deg)
    d = dfull[:, 0:1]
    y = lax.dot_general(x_ref[0], w_ref[...], (((1,), (1,)), ((), ())),
                        preferred_element_type=jnp.float32)
    y2_ref[0] = y * d
    d_ref[...] = d


def _fin_body(acc_ref, d_ref, b_ref, o_ref):
    o_ref[0] = acc_ref[0] * d_ref[...] + b_ref[...]


def kernel(x, src, dst, W, b):
    B, N, F_IN = x.shape
    F_OUT = W.shape[0]
    E = src.shape[0]

    npad = -(-N // (NS * ZR)) * (NS * ZR)

    degp = _make_deg(E, npad)(dst)
    degp = degp.reshape(NC, npad, LANES)

    R = 1000
    y2, d = pl.pallas_call(
        _prep_body,
        grid=(B, N // R),
        in_specs=[
            pl.BlockSpec((1, R, F_IN), lambda bb, i: (bb, i, 0)),
            pl.BlockSpec((F_OUT, F_IN), lambda bb, i: (0, 0)),
            pl.BlockSpec((NC, R, LANES), lambda bb, i: (0, i, 0)),
        ],
        out_specs=[
            pl.BlockSpec((1, R, F_OUT), lambda bb, i: (bb, i, 0)),
            pl.BlockSpec((R, 1), lambda bb, i: (i, 0)),
        ],
        out_shape=[
            jax.ShapeDtypeStruct((B, npad, F_OUT), jnp.float32),
            jax.ShapeDtypeStruct((N, 1), jnp.float32),
        ],
    )(x, W, degp)

    y2f = y2.reshape(B * npad, F_OUT)
    accf = _make_agg(E, npad, F_OUT)(y2f, src, dst)
    acc = accf.reshape(B, npad, F_OUT)[:, :N, :]

    out = pl.pallas_call(
        _fin_body,
        grid=(B, N // R),
        in_specs=[
            pl.BlockSpec((1, R, F_OUT), lambda bb, i: (bb, i, 0)),
            pl.BlockSpec((R, 1), lambda bb, i: (i, 0)),
            pl.BlockSpec((1, F_OUT), lambda bb, i: (0, 0)),
        ],
        out_specs=pl.BlockSpec((1, R, F_OUT), lambda bb, i: (bb, i, 0)),
        out_shape=jax.ShapeDtypeStruct((B, N, F_OUT), jnp.float32),
    )(acc, d, b.reshape(1, F_OUT))
    return out


# speedup vs baseline: 67.3997x; 1.3086x over previous
import functools

import jax
import jax.numpy as jnp
from jax import lax
from jax.experimental import pallas as pl
from jax.experimental.pallas import tpu as pltpu
from jax.experimental.pallas import tpu_sc as plsc

NC = 2
NS = 16
LANES = 16
CH = 80
ZR = 128


def _make_deg(E, N):
    ept = E // (NC * NS)
    n_chunks = ept // CH
    npt = N // NS
    nz = npt // ZR
    mesh = plsc.VectorSubcoreMesh(core_axis_name="c", subcore_axis_name="s",
                                  num_cores=NC, num_subcores=NS)

    @functools.partial(
        pl.kernel,
        out_type=jax.ShapeDtypeStruct((NC * N, LANES), jnp.float32),
        mesh=mesh,
        scratch_types=[
            pltpu.VMEM((CH,), jnp.int32),
            pltpu.VMEM((CH, LANES), jnp.float32),
            pltpu.VMEM((ZR, LANES), jnp.float32),
            pltpu.VMEM_SHARED((N, LANES), jnp.float32),
        ],
    )
    def deg_k(dst_hbm, out_hbm, dbuf, ones_v, zeros_v, acc_sh):
        c = lax.axis_index("c")
        s = lax.axis_index("s")
        wid = c * NS + s
        one16 = jnp.ones((LANES,), jnp.float32)
        zero16 = jnp.zeros((LANES,), jnp.float32)
        for i in range(CH):
            ones_v[i, :] = one16
        for i in range(ZR):
            zeros_v[i, :] = zero16

        rbase = s * npt

        def zbody(j, carry):
            pltpu.sync_copy(zeros_v, acc_sh.at[pl.ds(rbase + j * ZR, ZR)])
            return carry
        lax.fori_loop(0, nz, zbody, 0)
        plsc.subcore_barrier()

        ebase = wid * ept

        def ebody(g, carry):
            pltpu.sync_copy(dst_hbm.at[pl.ds(ebase + g * CH, CH)], dbuf)
            pltpu.sync_copy(ones_v, acc_sh.at[dbuf], add=True)
            return carry
        lax.fori_loop(0, n_chunks, ebody, 0)
        plsc.subcore_barrier()

        def obody(j, carry):
            r = rbase + j * ZR
            pltpu.sync_copy(acc_sh.at[pl.ds(r, ZR)],
                            out_hbm.at[pl.ds(c * N + r, ZR)])
            return carry
        lax.fori_loop(0, nz, obody, 0)

    return deg_k


def _make_agg(E, N, F):
    ept = E // NS
    n_chunks = ept // CH
    n_pairs = n_chunks // 2
    npt = N // NS
    no = npt // ZR
    mesh = plsc.VectorSubcoreMesh(core_axis_name="c", subcore_axis_name="s",
                                  num_cores=NC, num_subcores=NS)

    @functools.partial(
        pl.kernel,
        out_type=jax.ShapeDtypeStruct((NC * N, F), jnp.float32),
        mesh=mesh,
        scratch_types=[
            pltpu.VMEM((CH,), jnp.int32),
            pltpu.VMEM((CH,), jnp.int32),
            pltpu.VMEM((CH,), jnp.int32),
            pltpu.VMEM((CH,), jnp.int32),
            pltpu.VMEM((CH,), jnp.int32),
            pltpu.VMEM((CH,), jnp.int32),
            pltpu.VMEM((CH, F), jnp.float32),
            pltpu.VMEM((CH, F), jnp.float32),
            pltpu.VMEM((ZR, F), jnp.float32),
            pltpu.VMEM_SHARED((N, F), jnp.float32),
            pltpu.SemaphoreType.DMA,
            pltpu.SemaphoreType.DMA,
        ],
    )
    def agg_k(y2_hbm, src_hbm, dst_hbm, out_hbm,
              dbuf0, dbuf1, gbuf0, gbuf1, sbuf0, sbuf1,
              rows0, rows1, bounce_v, acc_sh, sem0, sem1):
        c = lax.axis_index("c")
        s = lax.axis_index("s")
        rbase = s * npt
        off = c * N

        def ibody(j, carry):
            r = rbase + j * ZR
            pltpu.sync_copy(y2_hbm.at[pl.ds(off + r, ZR)], bounce_v)
            pltpu.sync_copy(bounce_v, acc_sh.at[pl.ds(r, ZR)])
            return carry
        lax.fori_loop(0, no, ibody, 0)
        plsc.subcore_barrier()

        ebase = s * ept

        def load_idx(e0, dbuf, gbuf, sbuf):
            pltpu.sync_copy(dst_hbm.at[pl.ds(e0, CH)], dbuf)
            pltpu.sync_copy(src_hbm.at[pl.ds(e0, CH)], sbuf)
            for i in range(CH // LANES):
                sl = pl.ds(i * LANES, LANES)
                gbuf[sl] = dbuf[sl] + off

        def ebody(t, carry):
            load_idx(ebase + 2 * t * CH, dbuf0, gbuf0, sbuf0)
            cp0 = pltpu.async_copy(y2_hbm.at[gbuf0], rows0, sem0)
            load_idx(ebase + (2 * t + 1) * CH, dbuf1, gbuf1, sbuf1)
            cp1 = pltpu.async_copy(y2_hbm.at[gbuf1], rows1, sem1)
            cp0.wait()
            pltpu.sync_copy(rows0, acc_sh.at[sbuf0], add=True)
            cp1.wait()
            pltpu.sync_copy(rows1, acc_sh.at[sbuf1], add=True)
            return carry
        lax.fori_loop(0, n_pairs, ebody, 0)
        plsc.subcore_barrier()

        def obody(j, carry):
            r = rbase + j * ZR
            pltpu.sync_copy(acc_sh.at[pl.ds(r, ZR)], bounce_v)
            pltpu.sync_copy(bounce_v, out_hbm.at[pl.ds(off + r, ZR)])
            return carry
        lax.fori_loop(0, no, obody, 0)

    return agg_k


def _prep_body(x_ref, w_ref, deg_ref, y2_ref, d_ref):
    deg = deg_ref[0] + deg_ref[1] + 1.0
    dfull = lax.rsqrt(deg)
    d = dfull[:, 0:1]
    y = lax.dot_general(x_ref[0], w_ref[...], (((1,), (1,)), ((), ())),
                        preferred_element_type=jnp.float32)
    y2_ref[0] = y * d
    d_ref[...] = d


def _fin_body(acc_ref, d_ref, b_ref, o_ref):
    o_ref[0] = acc_ref[0] * d_ref[...] + b_ref[...]


def kernel(x, src, dst, W, b):
    B, N, F_IN = x.shape
    F_OUT = W.shape[0]
    E = src.shape[0]

    npad = -(-N // (NS * ZR)) * (NS * ZR)

    degp = _make_deg(E, npad)(dst)
    degp = degp.reshape(NC, npad, LANES)

    R = 1000
    y2, d = pl.pallas_call(
        _prep_body,
        grid=(B, N // R),
        in_specs=[
            pl.BlockSpec((1, R, F_IN), lambda bb, i: (bb, i, 0)),
            pl.BlockSpec((F_OUT, F_IN), lambda bb, i: (0, 0)),
            pl.BlockSpec((NC, R, LANES), lambda bb, i: (0, i, 0)),
        ],
        out_specs=[
            pl.BlockSpec((1, R, F_OUT), lambda bb, i: (bb, i, 0)),
            pl.BlockSpec((R, 1), lambda bb, i: (i, 0)),
        ],
        out_shape=[
            jax.ShapeDtypeStruct((B, npad, F_OUT), jnp.float32),
            jax.ShapeDtypeStruct((N, 1), jnp.float32),
        ],
    )(x, W, degp)

    y2f = y2.reshape(B * npad, F_OUT)
    accf = _make_agg(E, npad, F_OUT)(y2f, src, dst)
    acc = accf.reshape(B, npad, F_OUT)[:, :N, :]

    out = pl.pallas_call(
        _fin_body,
        grid=(B, N // R),
        in_specs=[
            pl.BlockSpec((1, R, F_OUT), lambda bb, i: (bb, i, 0)),
            pl.BlockSpec((R, 1), lambda bb, i: (i, 0)),
            pl.BlockSpec((1, F_OUT), lambda bb, i: (0, 0)),
        ],
        out_specs=pl.BlockSpec((1, R, F_OUT), lambda bb, i: (bb, i, 0)),
        out_shape=jax.ShapeDtypeStruct((B, N, F_OUT), jnp.float32),
    )(acc, d, b.reshape(1, F_OUT))
    return out


# speedup vs baseline: 74.4798x; 1.1050x over previous
import functools

import jax
import jax.numpy as jnp
from jax import lax
from jax.experimental import pallas as pl
from jax.experimental.pallas import tpu as pltpu
from jax.experimental.pallas import tpu_sc as plsc

NC = 2
NS = 16
LANES = 16
CH = 80
ZR = 128


def _make_deg(E, N):
    ept = E // (NC * NS)
    n_chunks = ept // CH
    npt = N // NS
    nz = npt // ZR
    mesh = plsc.VectorSubcoreMesh(core_axis_name="c", subcore_axis_name="s",
                                  num_cores=NC, num_subcores=NS)

    @functools.partial(
        pl.kernel,
        out_type=jax.ShapeDtypeStruct((NC * N, LANES), jnp.float32),
        mesh=mesh,
        scratch_types=[
            pltpu.VMEM((CH,), jnp.int32),
            pltpu.VMEM((CH, LANES), jnp.float32),
            pltpu.VMEM((ZR, LANES), jnp.float32),
            pltpu.VMEM_SHARED((N, LANES), jnp.float32),
        ],
    )
    def deg_k(dst_hbm, out_hbm, dbuf, ones_v, zeros_v, acc_sh):
        c = lax.axis_index("c")
        s = lax.axis_index("s")
        wid = c * NS + s
        one16 = jnp.ones((LANES,), jnp.float32)
        zero16 = jnp.zeros((LANES,), jnp.float32)
        for i in range(CH):
            ones_v[i, :] = one16
        for i in range(ZR):
            zeros_v[i, :] = zero16

        rbase = s * npt

        def zbody(j, carry):
            pltpu.sync_copy(zeros_v, acc_sh.at[pl.ds(rbase + j * ZR, ZR)])
            return carry
        lax.fori_loop(0, nz, zbody, 0)
        plsc.subcore_barrier()

        ebase = wid * ept

        def ebody(g, carry):
            pltpu.sync_copy(dst_hbm.at[pl.ds(ebase + g * CH, CH)], dbuf)
            pltpu.sync_copy(ones_v, acc_sh.at[dbuf], add=True)
            return carry
        lax.fori_loop(0, n_chunks, ebody, 0)
        plsc.subcore_barrier()

        def obody(j, carry):
            r = rbase + j * ZR
            pltpu.sync_copy(acc_sh.at[pl.ds(r, ZR)],
                            out_hbm.at[pl.ds(c * N + r, ZR)])
            return carry
        lax.fori_loop(0, nz, obody, 0)

    return deg_k


def _make_agg(E, N, F):
    ept = E // NS
    n_chunks = ept // CH
    n_pairs = n_chunks // 2
    npt = N // NS
    no = npt // ZR
    mesh = plsc.VectorSubcoreMesh(core_axis_name="c", subcore_axis_name="s",
                                  num_cores=NC, num_subcores=NS)

    @functools.partial(
        pl.kernel,
        out_type=jax.ShapeDtypeStruct((NC * N, F), jnp.float32),
        mesh=mesh,
        scratch_types=[
            [pltpu.VMEM((CH,), jnp.int32) for _ in range(3)],
            [pltpu.VMEM((CH,), jnp.int32) for _ in range(3)],
            [pltpu.VMEM((CH,), jnp.int32) for _ in range(3)],
            [pltpu.VMEM((CH, F), jnp.float32) for _ in range(3)],
            pltpu.VMEM_SHARED((N, F), jnp.float32),
            [pltpu.SemaphoreType.DMA for _ in range(3)],
            [pltpu.SemaphoreType.DMA for _ in range(3)],
        ],
    )
    def agg_k(y2_hbm, src_hbm, dst_hbm, out_hbm,
              dbufs, gbufs, sbufs, rows, acc_sh, gsems, ssems):
        c = lax.axis_index("c")
        s = lax.axis_index("s")
        rbase = s * npt
        off = c * N
        BR = CH

        def ibody(j, carry):
            r = rbase + j * BR
            pltpu.sync_copy(y2_hbm.at[pl.ds(off + r, BR)], rows[0])
            pltpu.sync_copy(rows[0], acc_sh.at[pl.ds(r, BR)])
            return carry
        lax.fori_loop(0, npt // BR, ibody, 0)
        plsc.subcore_barrier()

        ebase = s * ept

        def load_idx(e0, dbuf, gbuf, sbuf):
            pltpu.sync_copy(dst_hbm.at[pl.ds(e0, CH)], dbuf)
            pltpu.sync_copy(src_hbm.at[pl.ds(e0, CH)], sbuf)
            for i in range(CH // LANES):
                sl = pl.ds(i * LANES, LANES)
                gbuf[sl] = dbuf[sl] + off

        K = 3
        n_trip = n_chunks // K
        n_rem = n_chunks - n_trip * K

        def run_group(e0, nk):
            gcps = []
            for k in range(nk):
                load_idx(e0 + k * CH, dbufs[k], gbufs[k], sbufs[k])
                gcps.append(pltpu.async_copy(y2_hbm.at[gbufs[k]],
                                             rows[k], gsems[k]))
            scps = []
            for k in range(nk):
                gcps[k].wait()
                scps.append(pltpu.async_copy(rows[k], acc_sh.at[sbufs[k]],
                                             ssems[k], add=True))
            for k in range(nk):
                scps[k].wait()

        def ebody(t, carry):
            run_group(ebase + t * (K * CH), K)
            return carry
        lax.fori_loop(0, n_trip, ebody, 0)
        if n_rem:
            run_group(ebase + n_trip * K * CH, n_rem)
        plsc.subcore_barrier()

        def obody(j, carry):
            r = rbase + j * BR
            pltpu.sync_copy(acc_sh.at[pl.ds(r, BR)], rows[0])
            pltpu.sync_copy(rows[0], out_hbm.at[pl.ds(off + r, BR)])
            return carry
        lax.fori_loop(0, npt // BR, obody, 0)

    return agg_k


def _prep_body(x_ref, w_ref, deg_ref, y2_ref, d_ref):
    deg = deg_ref[0] + deg_ref[1] + 1.0
    dfull = lax.rsqrt(deg)
    d = dfull[:, 0:1]
    y = lax.dot_general(x_ref[0], w_ref[...], (((1,), (1,)), ((), ())),
                        preferred_element_type=jnp.float32)
    y2_ref[0] = y * d
    d_ref[...] = d


def _fin_body(acc_ref, d_ref, b_ref, o_ref):
    o_ref[0] = acc_ref[0] * d_ref[...] + b_ref[...]


def kernel(x, src, dst, W, b):
    B, N, F_IN = x.shape
    F_OUT = W.shape[0]
    E = src.shape[0]

    npad = -(-N // (NS * ZR)) * (NS * ZR)

    degp = _make_deg(E, npad)(dst)
    degp = degp.reshape(NC, npad, LANES)

    R = 1000
    y2, d = pl.pallas_call(
        _prep_body,
        grid=(B, N // R),
        in_specs=[
            pl.BlockSpec((1, R, F_IN), lambda bb, i: (bb, i, 0)),
            pl.BlockSpec((F_OUT, F_IN), lambda bb, i: (0, 0)),
            pl.BlockSpec((NC, R, LANES), lambda bb, i: (0, i, 0)),
        ],
        out_specs=[
            pl.BlockSpec((1, R, F_OUT), lambda bb, i: (bb, i, 0)),
            pl.BlockSpec((R, 1), lambda bb, i: (i, 0)),
        ],
        out_shape=[
            jax.ShapeDtypeStruct((B, npad, F_OUT), jnp.float32),
            jax.ShapeDtypeStruct((N, 1), jnp.float32),
        ],
    )(x, W, degp)

    y2f = y2.reshape(B * npad, F_OUT)
    accf = _make_agg(E, npad, F_OUT)(y2f, src, dst)
    acc = accf.reshape(B, npad, F_OUT)[:, :N, :]

    out = pl.pallas_call(
        _fin_body,
        grid=(B, N // R),
        in_specs=[
            pl.BlockSpec((1, R, F_OUT), lambda bb, i: (bb, i, 0)),
            pl.BlockSpec((R, 1), lambda bb, i: (i, 0)),
            pl.BlockSpec((1, F_OUT), lambda bb, i: (0, 0)),
        ],
        out_specs=pl.BlockSpec((1, R, F_OUT), lambda bb, i: (bb, i, 0)),
        out_shape=jax.ShapeDtypeStruct((B, N, F_OUT), jnp.float32),
    )(acc, d, b.reshape(1, F_OUT))
    return out


# speedup vs baseline: 79.4698x; 1.0670x over previous
import functools

import jax
import jax.numpy as jnp
from jax import lax
from jax.experimental import pallas as pl
from jax.experimental.pallas import tpu as pltpu
from jax.experimental.pallas import tpu_sc as plsc

NC = 2
NS = 16
LANES = 16
CH = 80
ZR = 128


def _make_deg(E, N):
    ept = E // (NC * NS)
    n_chunks = ept // CH
    npt = N // NS
    nz = npt // ZR
    mesh = plsc.VectorSubcoreMesh(core_axis_name="c", subcore_axis_name="s",
                                  num_cores=NC, num_subcores=NS)

    @functools.partial(
        pl.kernel,
        out_type=jax.ShapeDtypeStruct((NC * N, LANES), jnp.float32),
        mesh=mesh,
        scratch_types=[
            pltpu.VMEM((CH,), jnp.int32),
            pltpu.VMEM((CH, LANES), jnp.float32),
            pltpu.VMEM((ZR, LANES), jnp.float32),
            pltpu.VMEM_SHARED((N, LANES), jnp.float32),
        ],
    )
    def deg_k(dst_hbm, out_hbm, dbuf, ones_v, zeros_v, acc_sh):
        c = lax.axis_index("c")
        s = lax.axis_index("s")
        wid = c * NS + s
        one16 = jnp.ones((LANES,), jnp.float32)
        zero16 = jnp.zeros((LANES,), jnp.float32)
        for i in range(CH):
            ones_v[i, :] = one16
        for i in range(ZR):
            zeros_v[i, :] = zero16

        rbase = s * npt

        def zbody(j, carry):
            pltpu.sync_copy(zeros_v, acc_sh.at[pl.ds(rbase + j * ZR, ZR)])
            return carry
        lax.fori_loop(0, nz, zbody, 0)
        plsc.subcore_barrier()

        ebase = wid * ept

        def ebody(g, carry):
            pltpu.sync_copy(dst_hbm.at[pl.ds(ebase + g * CH, CH)], dbuf)
            pltpu.sync_copy(ones_v, acc_sh.at[dbuf], add=True)
            return carry
        lax.fori_loop(0, n_chunks, ebody, 0)
        plsc.subcore_barrier()

        def obody(j, carry):
            r = rbase + j * ZR
            pltpu.sync_copy(acc_sh.at[pl.ds(r, ZR)],
                            out_hbm.at[pl.ds(c * N + r, ZR)])
            return carry
        lax.fori_loop(0, nz, obody, 0)

    return deg_k


def _make_agg(E, N, F):
    ept = E // NS
    n_chunks = ept // CH
    n_pairs = n_chunks // 2
    npt = N // NS
    no = npt // ZR
    mesh = plsc.VectorSubcoreMesh(core_axis_name="c", subcore_axis_name="s",
                                  num_cores=NC, num_subcores=NS)

    @functools.partial(
        pl.kernel,
        out_type=jax.ShapeDtypeStruct((NC * N, F), jnp.float32),
        mesh=mesh,
        scratch_types=[
            [pltpu.VMEM((CH,), jnp.int32) for _ in range(3)],
            [pltpu.VMEM((CH,), jnp.int32) for _ in range(3)],
            [pltpu.VMEM((CH,), jnp.int32) for _ in range(3)],
            [pltpu.VMEM((CH, F), jnp.float32) for _ in range(3)],
            pltpu.VMEM_SHARED((N, F), jnp.float32),
            [pltpu.SemaphoreType.DMA for _ in range(3)],
            [pltpu.SemaphoreType.DMA for _ in range(3)],
        ],
    )
    def agg_k(y2_hbm, src_hbm, dst_hbm, out_hbm,
              dbufs, gbufs, sbufs, rows, acc_sh, gsems, ssems):
        c = lax.axis_index("c")
        s = lax.axis_index("s")
        rbase = s * npt
        off = c * N
        BR = CH

        def ibody(j, carry):
            r = rbase + j * BR
            pltpu.sync_copy(y2_hbm.at[pl.ds(off + r, BR)], rows[0])
            pltpu.sync_copy(rows[0], acc_sh.at[pl.ds(r, BR)])
            return carry
        lax.fori_loop(0, npt // BR, ibody, 0)
        plsc.subcore_barrier()

        ebase = s * ept

        def load_idx(e0, dbuf, gbuf, sbuf):
            pltpu.sync_copy(dst_hbm.at[pl.ds(e0, CH)], dbuf)
            pltpu.sync_copy(src_hbm.at[pl.ds(e0, CH)], sbuf)
            for i in range(CH // LANES):
                sl = pl.ds(i * LANES, LANES)
                gbuf[sl] = dbuf[sl] + off

        K = 3
        n_trip = n_chunks // K
        n_rem = n_chunks - n_trip * K
        dummy = y2_hbm.at[pl.ds(0, CH)]

        def ebody(t, carry):
            gcps = []
            for k in range(K):
                @pl.when(t > 0)
                def _():
                    pltpu.make_async_copy(dummy, rows[k], ssems[k]).wait()
                load_idx(ebase + (t * K + k) * CH, dbufs[k], gbufs[k],
                         sbufs[k])
                gcps.append(pltpu.async_copy(y2_hbm.at[gbufs[k]],
                                             rows[k], gsems[k]))
            for k in range(K):
                gcps[k].wait()
                pltpu.async_copy(rows[k], acc_sh.at[sbufs[k]],
                                 ssems[k], add=True)
            return carry
        lax.fori_loop(0, n_trip, ebody, 0)
        for k in range(K):
            pltpu.make_async_copy(dummy, rows[k], ssems[k]).wait()
        if n_rem:
            gcps = []
            for k in range(n_rem):
                load_idx(ebase + (n_trip * K + k) * CH, dbufs[k], gbufs[k],
                         sbufs[k])
                gcps.append(pltpu.async_copy(y2_hbm.at[gbufs[k]],
                                             rows[k], gsems[k]))
            for k in range(n_rem):
                gcps[k].wait()
                cp = pltpu.async_copy(rows[k], acc_sh.at[sbufs[k]],
                                      ssems[k], add=True)
                cp.wait()
        plsc.subcore_barrier()

        def obody(j, carry):
            r = rbase + j * BR
            pltpu.sync_copy(acc_sh.at[pl.ds(r, BR)], rows[0])
            pltpu.sync_copy(rows[0], out_hbm.at[pl.ds(off + r, BR)])
            return carry
        lax.fori_loop(0, npt // BR, obody, 0)

    return agg_k


def _prep_body(x_ref, w_ref, deg_ref, y2_ref, d_ref):
    deg = deg_ref[0] + deg_ref[1] + 1.0
    dfull = lax.rsqrt(deg)
    d = dfull[:, 0:1]
    y = lax.dot_general(x_ref[0], w_ref[...], (((1,), (1,)), ((), ())),
                        preferred_element_type=jnp.float32)
    y2_ref[0] = y * d
    d_ref[...] = d


def _fin_body(acc_ref, d_ref, b_ref, o_ref):
    o_ref[0] = acc_ref[0] * d_ref[...] + b_ref[...]


def kernel(x, src, dst, W, b):
    B, N, F_IN = x.shape
    F_OUT = W.shape[0]
    E = src.shape[0]

    npad = -(-N // (NS * ZR)) * (NS * ZR)

    degp = _make_deg(E, npad)(dst)
    degp = degp.reshape(NC, npad, LANES)

    R = 1000
    y2, d = pl.pallas_call(
        _prep_body,
        grid=(B, N // R),
        in_specs=[
            pl.BlockSpec((1, R, F_IN), lambda bb, i: (bb, i, 0)),
            pl.BlockSpec((F_OUT, F_IN), lambda bb, i: (0, 0)),
            pl.BlockSpec((NC, R, LANES), lambda bb, i: (0, i, 0)),
        ],
        out_specs=[
            pl.BlockSpec((1, R, F_OUT), lambda bb, i: (bb, i, 0)),
            pl.BlockSpec((R, 1), lambda bb, i: (i, 0)),
        ],
        out_shape=[
            jax.ShapeDtypeStruct((B, npad, F_OUT), jnp.float32),
            jax.ShapeDtypeStruct((N, 1), jnp.float32),
        ],
    )(x, W, degp)

    y2f = y2.reshape(B * npad, F_OUT)
    accf = _make_agg(E, npad, F_OUT)(y2f, src, dst)
    acc = accf.reshape(B, npad, F_OUT)[:, :N, :]

    out = pl.pallas_call(
        _fin_body,
        grid=(B, N // R),
        in_specs=[
            pl.BlockSpec((1, R, F_OUT), lambda bb, i: (bb, i, 0)),
            pl.BlockSpec((R, 1), lambda bb, i: (i, 0)),
            pl.BlockSpec((1, F_OUT), lambda bb, i: (0, 0)),
        ],
        out_specs=pl.BlockSpec((1, R, F_OUT), lambda bb, i: (bb, i, 0)),
        out_shape=jax.ShapeDtypeStruct((B, N, F_OUT), jnp.float32),
    )(acc, d, b.reshape(1, F_OUT))
    return out


# speedup vs baseline: 91.2755x; 1.1486x over previous
import functools

import jax
import jax.numpy as jnp
from jax import lax
from jax.experimental import pallas as pl
from jax.experimental.pallas import tpu as pltpu
from jax.experimental.pallas import tpu_sc as plsc

NC = 2
NS = 16
LANES = 16
CH = 80
ZR = 128


def _make_deg(E, N):
    ept = E // (NC * NS)
    n_chunks = ept // CH
    npt = N // NS
    nz = npt // ZR
    mesh = plsc.VectorSubcoreMesh(core_axis_name="c", subcore_axis_name="s",
                                  num_cores=NC, num_subcores=NS)

    @functools.partial(
        pl.kernel,
        out_type=jax.ShapeDtypeStruct((NC * N, LANES), jnp.float32),
        mesh=mesh,
        scratch_types=[
            pltpu.VMEM((CH,), jnp.int32),
            pltpu.VMEM((CH, LANES), jnp.float32),
            pltpu.VMEM((ZR, LANES), jnp.float32),
            pltpu.VMEM_SHARED((N, LANES), jnp.float32),
        ],
    )
    def deg_k(dst_hbm, out_hbm, dbuf, ones_v, zeros_v, acc_sh):
        c = lax.axis_index("c")
        s = lax.axis_index("s")
        wid = c * NS + s
        one16 = jnp.ones((LANES,), jnp.float32)
        zero16 = jnp.zeros((LANES,), jnp.float32)
        for i in range(CH):
            ones_v[i, :] = one16
        for i in range(ZR):
            zeros_v[i, :] = zero16

        rbase = s * npt

        def zbody(j, carry):
            pltpu.sync_copy(zeros_v, acc_sh.at[pl.ds(rbase + j * ZR, ZR)])
            return carry
        lax.fori_loop(0, nz, zbody, 0)
        plsc.subcore_barrier()

        ebase = wid * ept

        def ebody(g, carry):
            pltpu.sync_copy(dst_hbm.at[pl.ds(ebase + g * CH, CH)], dbuf)
            pltpu.sync_copy(ones_v, acc_sh.at[dbuf], add=True)
            return carry
        lax.fori_loop(0, n_chunks, ebody, 0)
        plsc.subcore_barrier()

        def obody(j, carry):
            r = rbase + j * ZR
            pltpu.sync_copy(acc_sh.at[pl.ds(r, ZR)],
                            out_hbm.at[pl.ds(c * N + r, ZR)])
            return carry
        lax.fori_loop(0, nz, obody, 0)

    return deg_k


def _make_agg(E, N, F):
    ept = E // NS
    n_chunks = ept // CH
    n_pairs = n_chunks // 2
    npt = N // NS
    no = npt // ZR
    mesh = plsc.VectorSubcoreMesh(core_axis_name="c", subcore_axis_name="s",
                                  num_cores=NC, num_subcores=NS)

    @functools.partial(
        pl.kernel,
        out_type=jax.ShapeDtypeStruct((NC * N, F), jnp.float32),
        mesh=mesh,
        scratch_types=[
            pltpu.VMEM((3 * CH,), jnp.int32),
            [pltpu.VMEM((CH,), jnp.int32) for _ in range(3)],
            [pltpu.VMEM((CH,), jnp.int32) for _ in range(3)],
            [pltpu.VMEM((CH, F), jnp.float32) for _ in range(3)],
            pltpu.VMEM_SHARED((N, F), jnp.float32),
            [pltpu.SemaphoreType.DMA for _ in range(3)],
            [pltpu.SemaphoreType.DMA for _ in range(3)],
        ],
    )
    def agg_k(y2_hbm, src_hbm, dst_hbm, out_hbm,
              dgrp, gbufs, sbufs, rows, acc_sh, gsems, ssems):
        c = lax.axis_index("c")
        s = lax.axis_index("s")
        rbase = s * npt
        off = c * N
        BR = CH

        def ibody(j, carry):
            r = rbase + j * BR
            pltpu.sync_copy(y2_hbm.at[pl.ds(off + r, BR)], rows[0])
            pltpu.sync_copy(rows[0], acc_sh.at[pl.ds(r, BR)])
            return carry
        lax.fori_loop(0, npt // BR, ibody, 0)
        plsc.subcore_barrier()

        ebase = s * ept

        def load_group(e0):
            pltpu.sync_copy(dst_hbm.at[pl.ds(e0, 3 * CH)], dgrp)

        def fill_slot(k, e0):
            for i in range(CH // LANES):
                o = k * CH + i * LANES
                sl = pl.ds(i * LANES, LANES)
                gbufs[k][sl] = dgrp[pl.ds(o, LANES)] + off
            pltpu.sync_copy(src_hbm.at[pl.ds(e0 + k * CH, CH)], sbufs[k])

        K = 3
        n_trip = n_chunks // K
        n_rem = n_chunks - n_trip * K
        dummy = y2_hbm.at[pl.ds(0, CH)]

        def ebody(t, carry):
            e0 = ebase + t * (K * CH)
            load_group(e0)
            gcps = []
            for k in range(K):
                @pl.when(t > 0)
                def _():
                    pltpu.make_async_copy(dummy, rows[k], ssems[k]).wait()
                fill_slot(k, e0)
                gcps.append(pltpu.async_copy(y2_hbm.at[gbufs[k]],
                                             rows[k], gsems[k]))
            for k in range(K):
                gcps[k].wait()
                pltpu.async_copy(rows[k], acc_sh.at[sbufs[k]],
                                 ssems[k], add=True)
            return carry
        lax.fori_loop(0, n_trip, ebody, 0)
        for k in range(K):
            pltpu.make_async_copy(dummy, rows[k], ssems[k]).wait()
        if n_rem:
            for k in range(n_rem):
                e1 = ebase + (n_trip * K + k) * CH
                pltpu.sync_copy(dst_hbm.at[pl.ds(e1, CH)],
                                dgrp.at[pl.ds(0, CH)])
                fill_slot(0, e1)
                pltpu.async_copy(y2_hbm.at[gbufs[0]],
                                 rows[0], gsems[0]).wait()
                pltpu.async_copy(rows[0], acc_sh.at[sbufs[0]],
                                 ssems[0], add=True).wait()
        plsc.subcore_barrier()

        def obody(j, carry):
            r = rbase + j * BR
            pltpu.sync_copy(acc_sh.at[pl.ds(r, BR)], rows[0])
            pltpu.sync_copy(rows[0], out_hbm.at[pl.ds(off + r, BR)])
            return carry
        lax.fori_loop(0, npt // BR, obody, 0)

    return agg_k


def _prep_body(x_ref, w_ref, deg_ref, y2_ref, d_ref):
    deg = deg_ref[0] + deg_ref[1] + 1.0
    dfull = lax.rsqrt(deg)
    d = dfull[:, 0:1]
    y = lax.dot_general(x_ref[0], w_ref[...], (((1,), (1,)), ((), ())),
                        preferred_element_type=jnp.float32)
    y2_ref[0] = y * d
    d_ref[...] = d


def _fin_body(acc_ref, d_ref, b_ref, o_ref):
    o_ref[0] = acc_ref[0] * d_ref[...] + b_ref[...]


def kernel(x, src, dst, W, b):
    B, N, F_IN = x.shape
    F_OUT = W.shape[0]
    E = src.shape[0]

    npad = -(-N // (NS * ZR)) * (NS * ZR)

    degp = _make_deg(E, npad)(dst)
    degp = degp.reshape(NC, npad, LANES)

    R = 1000
    y2, d = pl.pallas_call(
        _prep_body,
        grid=(B, N // R),
        in_specs=[
            pl.BlockSpec((1, R, F_IN), lambda bb, i: (bb, i, 0)),
            pl.BlockSpec((F_OUT, F_IN), lambda bb, i: (0, 0)),
            pl.BlockSpec((NC, R, LANES), lambda bb, i: (0, i, 0)),
        ],
        out_specs=[
            pl.BlockSpec((1, R, F_OUT), lambda bb, i: (bb, i, 0)),
            pl.BlockSpec((R, 1), lambda bb, i: (i, 0)),
        ],
        out_shape=[
            jax.ShapeDtypeStruct((B, npad, F_OUT), jnp.float32),
            jax.ShapeDtypeStruct((N, 1), jnp.float32),
        ],
    )(x, W, degp)

    y2f = y2.reshape(B * npad, F_OUT)
    accf = _make_agg(E, npad, F_OUT)(y2f, src, dst)
    acc = accf.reshape(B, npad, F_OUT)[:, :N, :]

    out = pl.pallas_call(
        _fin_body,
        grid=(B, N // R),
        in_specs=[
            pl.BlockSpec((1, R, F_OUT), lambda bb, i: (bb, i, 0)),
            pl.BlockSpec((R, 1), lambda bb, i: (i, 0)),
            pl.BlockSpec((1, F_OUT), lambda bb, i: (0, 0)),
        ],
        out_specs=pl.BlockSpec((1, R, F_OUT), lambda bb, i: (bb, i, 0)),
        out_shape=jax.ShapeDtypeStruct((B, N, F_OUT), jnp.float32),
    )(acc, d, b.reshape(1, F_OUT))
    return out


# speedup vs baseline: 96.1880x; 1.0538x over previous
import functools

import jax
import jax.numpy as jnp
from jax import lax
from jax.experimental import pallas as pl
from jax.experimental.pallas import tpu as pltpu
from jax.experimental.pallas import tpu_sc as plsc

NC = 2
NS = 16
LANES = 16
CH = 80
ZR = 128


def _make_deg(E, N):
    ept = E // (NC * NS)
    n_chunks = ept // CH
    npt = N // NS
    nz = npt // ZR
    mesh = plsc.VectorSubcoreMesh(core_axis_name="c", subcore_axis_name="s",
                                  num_cores=NC, num_subcores=NS)

    @functools.partial(
        pl.kernel,
        out_type=jax.ShapeDtypeStruct((NC * N, LANES), jnp.float32),
        mesh=mesh,
        scratch_types=[
            pltpu.VMEM((CH,), jnp.int32),
            pltpu.VMEM((CH, LANES), jnp.float32),
            pltpu.VMEM((ZR, LANES), jnp.float32),
            pltpu.VMEM_SHARED((N, LANES), jnp.float32),
        ],
    )
    def deg_k(dst_hbm, out_hbm, dbuf, ones_v, zeros_v, acc_sh):
        c = lax.axis_index("c")
        s = lax.axis_index("s")
        wid = c * NS + s
        one16 = jnp.ones((LANES,), jnp.float32)
        zero16 = jnp.zeros((LANES,), jnp.float32)
        for i in range(CH):
            ones_v[i, :] = one16
        for i in range(ZR):
            zeros_v[i, :] = zero16

        rbase = s * npt

        def zbody(j, carry):
            pltpu.sync_copy(zeros_v, acc_sh.at[pl.ds(rbase + j * ZR, ZR)])
            return carry
        lax.fori_loop(0, nz, zbody, 0)
        plsc.subcore_barrier()

        ebase = wid * ept

        def ebody(g, carry):
            pltpu.sync_copy(dst_hbm.at[pl.ds(ebase + g * CH, CH)], dbuf)
            pltpu.sync_copy(ones_v, acc_sh.at[dbuf], add=True)
            return carry
        lax.fori_loop(0, n_chunks, ebody, 0)
        plsc.subcore_barrier()

        def obody(j, carry):
            r = rbase + j * ZR
            pltpu.sync_copy(acc_sh.at[pl.ds(r, ZR)],
                            out_hbm.at[pl.ds(c * N + r, ZR)])
            return carry
        lax.fori_loop(0, nz, obody, 0)

    return deg_k


def _make_agg(E, N, F):
    ept = E // NS
    n_chunks = ept // CH
    n_pairs = n_chunks // 2
    npt = N // NS
    no = npt // ZR
    mesh = plsc.VectorSubcoreMesh(core_axis_name="c", subcore_axis_name="s",
                                  num_cores=NC, num_subcores=NS)

    @functools.partial(
        pl.kernel,
        out_type=jax.ShapeDtypeStruct((NC * N, F), jnp.float32),
        mesh=mesh,
        scratch_types=[
            pltpu.VMEM((3 * CH,), jnp.int32),
            [pltpu.VMEM((CH,), jnp.int32) for _ in range(3)],
            [pltpu.VMEM((CH, F), jnp.float32) for _ in range(3)],
            pltpu.VMEM_SHARED((N, F), jnp.float32),
            [pltpu.SemaphoreType.DMA for _ in range(3)],
            [pltpu.SemaphoreType.DMA for _ in range(3)],
            [pltpu.SemaphoreType.DMA for _ in range(3)],
        ],
    )
    def agg_k(y2_hbm, src_hbm, dst2_hbm, out_hbm,
              dgrp, sbufs, rows, acc_sh, gsems, ssems, isems):
        c = lax.axis_index("c")
        s = lax.axis_index("s")
        rbase = s * npt
        off = c * N
        eoff = c * (NS * ept)
        BR = CH

        def ibody(j, carry):
            r = rbase + j * BR
            pltpu.sync_copy(y2_hbm.at[pl.ds(off + r, BR)], rows[0])
            pltpu.sync_copy(rows[0], acc_sh.at[pl.ds(r, BR)])
            return carry
        lax.fori_loop(0, npt // BR, ibody, 0)
        plsc.subcore_barrier()

        ebase = s * ept


        K = 3
        n_trip = n_chunks // K
        n_rem = n_chunks - n_trip * K
        dummy = y2_hbm.at[pl.ds(0, CH)]

        def ebody(t, carry):
            e0 = ebase + t * (K * CH)
            pltpu.sync_copy(dst2_hbm.at[pl.ds(eoff + e0, K * CH)], dgrp)
            icps, gcps = [], []
            for k in range(K):
                @pl.when(t > 0)
                def _():
                    pltpu.make_async_copy(dummy, rows[k], ssems[k]).wait()
                icps.append(pltpu.async_copy(
                    src_hbm.at[pl.ds(e0 + k * CH, CH)], sbufs[k], isems[k]))
                gcps.append(pltpu.async_copy(
                    y2_hbm.at[dgrp.at[pl.ds(k * CH, CH)]], rows[k],
                    gsems[k]))
            for k in range(K):
                gcps[k].wait()
                icps[k].wait()
                pltpu.async_copy(rows[k], acc_sh.at[sbufs[k]],
                                 ssems[k], add=True)
            return carry
        lax.fori_loop(0, n_trip, ebody, 0)
        for k in range(K):
            pltpu.make_async_copy(dummy, rows[k], ssems[k]).wait()
        if n_rem:
            for k in range(n_rem):
                e1 = ebase + (n_trip * K + k) * CH
                pltpu.sync_copy(dst2_hbm.at[pl.ds(eoff + e1, CH)],
                                dgrp.at[pl.ds(0, CH)])
                pltpu.sync_copy(src_hbm.at[pl.ds(e1, CH)], sbufs[0])
                pltpu.async_copy(y2_hbm.at[dgrp.at[pl.ds(0, CH)]],
                                 rows[0], gsems[0]).wait()
                pltpu.async_copy(rows[0], acc_sh.at[sbufs[0]],
                                 ssems[0], add=True).wait()
        plsc.subcore_barrier()

        def obody(j, carry):
            r = rbase + j * BR
            pltpu.sync_copy(acc_sh.at[pl.ds(r, BR)], rows[0])
            pltpu.sync_copy(rows[0], out_hbm.at[pl.ds(off + r, BR)])
            return carry
        lax.fori_loop(0, npt // BR, obody, 0)

    return agg_k


def _prep_body(x_ref, w_ref, deg_ref, y2_ref, d_ref):
    deg = deg_ref[0] + deg_ref[1] + 1.0
    dfull = lax.rsqrt(deg)
    d = dfull[:, 0:1]
    y = lax.dot_general(x_ref[0], w_ref[...], (((1,), (1,)), ((), ())),
                        preferred_element_type=jnp.float32)
    y2_ref[0] = y * d
    d_ref[...] = d


def _make_idx_body(npad):
    def _idx_body(dst_ref, o_ref):
        o_ref[0] = dst_ref[...]
        o_ref[1] = dst_ref[...] + npad
    return _idx_body


def _fin_body(acc_ref, d_ref, b_ref, o_ref):
    o_ref[0] = acc_ref[0] * d_ref[...] + b_ref[...]


def kernel(x, src, dst, W, b):
    B, N, F_IN = x.shape
    F_OUT = W.shape[0]
    E = src.shape[0]

    npad = -(-N // (NS * ZR)) * (NS * ZR)

    degp = _make_deg(E, npad)(dst)
    degp = degp.reshape(NC, npad, LANES)

    R = 1000
    y2, d = pl.pallas_call(
        _prep_body,
        grid=(B, N // R),
        in_specs=[
            pl.BlockSpec((1, R, F_IN), lambda bb, i: (bb, i, 0)),
            pl.BlockSpec((F_OUT, F_IN), lambda bb, i: (0, 0)),
            pl.BlockSpec((NC, R, LANES), lambda bb, i: (0, i, 0)),
        ],
        out_specs=[
            pl.BlockSpec((1, R, F_OUT), lambda bb, i: (bb, i, 0)),
            pl.BlockSpec((R, 1), lambda bb, i: (i, 0)),
        ],
        out_shape=[
            jax.ShapeDtypeStruct((B, npad, F_OUT), jnp.float32),
            jax.ShapeDtypeStruct((N, 1), jnp.float32),
        ],
    )(x, W, degp)

    dst2 = pl.pallas_call(
        _make_idx_body(npad),
        grid=(1,),
        in_specs=[pl.BlockSpec((E // 128, 128), lambda i: (0, 0))],
        out_specs=pl.BlockSpec((NC, E // 128, 128), lambda i: (0, 0, 0)),
        out_shape=jax.ShapeDtypeStruct((NC, E // 128, 128), jnp.int32),
    )(dst.reshape(E // 128, 128)).reshape(NC * E)

    y2f = y2.reshape(B * npad, F_OUT)
    accf = _make_agg(E, npad, F_OUT)(y2f, src, dst2)
    acc = accf.reshape(B, npad, F_OUT)[:, :N, :]

    out = pl.pallas_call(
        _fin_body,
        grid=(B, N // R),
        in_specs=[
            pl.BlockSpec((1, R, F_OUT), lambda bb, i: (bb, i, 0)),
            pl.BlockSpec((R, 1), lambda bb, i: (i, 0)),
            pl.BlockSpec((1, F_OUT), lambda bb, i: (0, 0)),
        ],
        out_specs=pl.BlockSpec((1, R, F_OUT), lambda bb, i: (bb, i, 0)),
        out_shape=jax.ShapeDtypeStruct((B, N, F_OUT), jnp.float32),
    )(acc, d, b.reshape(1, F_OUT))
    return out
